# baseline (speedup 1.0000x reference)
import functools

import jax
import jax.numpy as jnp
from jax import lax
from jax.experimental import pallas as pl
from jax.experimental.pallas import tpu as pltpu
from jax.experimental.pallas import tpu_sc as plsc

BATCH = 16384
DIM = 64
VOCAB = 1000000
MAIN_V = 999424
STRIPE = MAIN_V // 32
TAIL = VOCAB - MAIN_V
CH = 512
N_MAIN = STRIPE // CH
N_WIN = N_MAIN + 2
N_GRP = BATCH // 16

_info = plsc.get_sparse_core_info()
_NC, _NS = _info.num_cores, _info.num_subcores

_mesh = plsc.VectorSubcoreMesh(core_axis_name="c", subcore_axis_name="s")


@functools.partial(
    pl.kernel,
    mesh=_mesh,
    out_type=jax.ShapeDtypeStruct((BATCH, 128), jnp.float32),
    scratch_types=[
        pltpu.VMEM((BATCH + 16,), jnp.int32),
        pltpu.VMEM((BATCH + 16,), jnp.int32),
        pltpu.VMEM((BATCH + 16,), jnp.int32),
        pltpu.VMEM((80,), jnp.int32),
        pltpu.VMEM((64,), jnp.int32),
        pltpu.VMEM((2, 8, 8, CH), jnp.float32),
        pltpu.VMEM((32, 128), jnp.float32),
        pltpu.VMEM((32,), jnp.int32),
        pltpu.SemaphoreType.DMA,
        pltpu.SemaphoreType.DMA,
        pltpu.SemaphoreType.DMA,
        pltpu.SemaphoreType.DMA,
    ],
    compiler_params=pltpu.CompilerParams(needs_layout_passes=False),
)
def _embed(idx_hbm, wt3_hbm, wtl3_hbm, out_hbm, idx_v, hits_v, sort_v,
           st_v, cur_v, buf_v, ext_v, bext_v,
           st_sem0, st_sem1, sc_sem0, sc_sem1):
    wid = lax.axis_index("s") * _NC + lax.axis_index("c")
    lo = wid * STRIPE
    iot = lax.iota(jnp.int32, 16)
    lane0 = iot == 0

    pltpu.sync_copy(idx_hbm.at[pl.ds(0, BATCH)], idx_v.at[pl.ds(0, BATCH)])

    sems = (st_sem0, st_sem1)

    def fire_window(t, p):
        @pl.when(t < N_MAIN)
        def _():
            off = lo + t * CH
            for g in range(8):
                pltpu.async_copy(
                    wt3_hbm.at[g, :, pl.ds(off, CH)], buf_v.at[p, g],
                    sems[p],
                )

        @pl.when(t >= N_MAIN)
        def _():
            off = (t - N_MAIN) * CH
            for g in range(8):
                pltpu.async_copy(
                    wtl3_hbm.at[g, :, pl.ds(off, CH)], buf_v.at[p, g],
                    sems[p],
                )

    def wait_window(p):
        for g in range(8):
            pltpu.make_async_copy(
                wt3_hbm.at[0, :, pl.ds(0, CH)], buf_v.at[p, 0], sems[p]
            ).wait()

    fire_window(0, 0)
    fire_window(1, 1)

    zeros = iot * 0
    for c in range(5):
        st_v[pl.ds(c * 16, 16)] = zeros

    def prepass(q, cur):
        iv = idx_v[pl.ds(q * 16, 16)]
        bv = q * 16 + iot
        m_main = (iv >= lo) & (iv < lo + STRIPE)
        m_tail = (iv >= MAIN_V) & ((bv & 31) == wid)
        rel = jnp.where(m_tail, iv - MAIN_V + STRIPE, iv - lo)
        m = m_main | m_tail
        mi = jnp.where(m, 1, 0)
        cnt = plsc.all_reduce_population_count(m)[0]

        @pl.when(cnt > 0)
        def _():
            packed = (rel << 14) | bv
            csum = jnp.cumsum(mi)
            pos = cur + csum - mi
            plsc.store_scatter(hits_v, [pos], packed, mask=m)
            w_v = lax.shift_right_logical(rel, 9) & 63
            plsc.addupdate_scatter(st_v, [w_v], mi, mask=m)

        return cur + cnt

    n_hits = lax.fori_loop(0, N_GRP, prepass, 0)
    n_hit_grp = (n_hits + 15) >> 4

    carry = 0
    for c in range(4):
        h = st_v[pl.ds(c * 16, 16)]
        cs = jnp.cumsum(h)
        ex = cs - h + carry
        st_v[pl.ds(c * 16, 16)] = ex
        cur_v[pl.ds(c * 16, 16)] = ex
        carry = carry + cs[15]
    st_v[pl.ds(64, 16)] = zeros + carry

    def place(q, _):
        hv = hits_v[pl.ds(q * 16, 16)]
        w_v = lax.shift_right_logical(hv, 23) & 63
        valid = (q * 16 + iot) < n_hits
        vi = jnp.where(valid, 1, 0)
        for l in range(16):
            @pl.when(vi[l] == 1)
            def _():
                wsp = iot * 0 + w_v[l]
                cur = plsc.load_gather(cur_v, [wsp])
                plsc.store_scatter(sort_v, [cur], iot * 0 + hv[l],
                                   mask=lane0)
                plsc.store_scatter(cur_v, [wsp], cur + 1, mask=lane0)
        return 0

    lax.fori_loop(0, n_hit_grp, place, 0)

    def get_start(w):
        chunk = st_v[pl.ds((w >> 4) * 16, 16)]
        return jnp.sum(jnp.where(iot == (w & 15), chunk, 0))

    def _wait_half(h, sem):
        bv = bext_v[pl.ds(h * 16, 16)]
        pltpu.make_async_copy(
            ext_v.at[pl.ds(h * 16, 16)], out_hbm.at[bv], sem
        ).wait()

    def _fire_half(h, sem):
        bv = bext_v[pl.ds(h * 16, 16)]
        pltpu.async_copy(ext_v.at[pl.ds(h * 16, 16)], out_hbm.at[bv], sem)

    def fire_due(j, st2):
        wf0, wf1 = st2

        @pl.when(((j & 1) == 0) & (wf0 == 1))
        def _():
            _wait_half(0, sc_sem0)

        @pl.when((j & 1) == 0)
        def _():
            _fire_half(0, sc_sem0)

        @pl.when(((j & 1) == 1) & (wf1 == 1))
        def _():
            _wait_half(1, sc_sem1)

        @pl.when((j & 1) == 1)
        def _():
            _fire_half(1, sc_sem1)

        wf0 = jnp.where((j & 1) == 0, 1, wf0)
        wf1 = jnp.where((j & 1) == 1, 1, wf1)
        return wf0, wf1

    def scan_window(t, p, state):
        tb, nf, wf0, wf1 = state
        start = get_start(t)
        end = get_start(t + 1)
        rlo = t * CH

        def grp(g, st2):
            tb, nf, wf0, wf1 = st2
            wf0, wf1 = lax.fori_loop(nf, g, fire_due, (wf0, wf1))
            nf = jnp.maximum(nf, g)
            first = g >= tb

            @pl.when(first & ((g & 1) == 0) & (wf0 == 1))
            def _():
                _wait_half(0, sc_sem0)

            @pl.when(first & ((g & 1) == 1) & (wf1 == 1))
            def _():
                _wait_half(1, sc_sem1)

            wf0 = jnp.where(first & ((g & 1) == 0), 0, wf0)
            wf1 = jnp.where(first & ((g & 1) == 1), 0, wf1)
            tb = jnp.maximum(tb, g + 1)

            posv = g * 16 + iot
            hv = sort_v[pl.ds(g * 16, 16)]
            m = (posv >= start) & (posv < end)
            rel = lax.shift_right_logical(hv, 14)
            bv = hv & 16383
            lrel = (rel - rlo) & (CH - 1)
            row_v = posv & 31
            for j in range(64):
                vals = plsc.load_gather(
                    buf_v.at[p],
                    [zeros + (j >> 3), zeros + (j & 7), lrel],
                    mask=m,
                )
                plsc.store_scatter(ext_v, [row_v, zeros + j], vals, mask=m)
            plsc.store_scatter(bext_v, [row_v], bv, mask=m)
            return tb, nf, wf0, wf1

        state2 = lax.fori_loop(start >> 4, (end + 15) >> 4, grp,
                               (tb, nf, wf0, wf1))
        tb, nf, wf0, wf1 = state2
        nf_new = end >> 4
        wf0, wf1 = lax.fori_loop(nf, nf_new, fire_due, (wf0, wf1))
        nf = jnp.maximum(nf, nf_new)
        return tb, nf, wf0, wf1

    def outer(k, state):
        for p in range(2):
            t = 2 * k + p
            wait_window(p)
            state = scan_window(t, p, state)

            @pl.when(t + 2 < N_WIN)
            def _():
                fire_window(t + 2, p)

        return state

    state = lax.fori_loop(0, (N_WIN - 1) // 2, outer, (0, 0, 0, 0))
    wait_window(0)
    tb, nf, wf0, wf1 = scan_window(N_WIN - 1, 0, state)

    slot = n_hits & 31
    rem = n_hits & 15
    half = (slot >> 4) & 1

    @pl.when((rem != 0) & (half == 0) & (wf0 == 1))
    def _():
        _wait_half(0, sc_sem0)

    @pl.when((rem != 0) & (half == 1) & (wf1 == 1))
    def _():
        _wait_half(1, sc_sem1)

    wf0 = jnp.where((rem != 0) & (half == 0), 0, wf0)
    wf1 = jnp.where((rem != 0) & (half == 1), 0, wf1)

    src_row = (n_hits - 1) & 31
    src_v = iot * 0 + src_row
    pb = plsc.load_gather(bext_v, [src_v])

    for r in range(1, 16):
        @pl.when((rem != 0) & (r >= rem))
        def _():
            row_v = iot * 0 + ((slot & 16) + r)
            for jq in range(8):
                vals = plsc.load_gather(ext_v, [src_v, jq * 16 + iot])
                plsc.store_scatter(ext_v, [row_v, jq * 16 + iot], vals)
            plsc.store_scatter(bext_v, [row_v], pb, mask=lane0)

    @pl.when((rem != 0) & (half == 0))
    def _():
        _fire_half(0, sc_sem0)

    @pl.when((rem != 0) & (half == 1))
    def _():
        _fire_half(1, sc_sem1)

    wf0 = jnp.where((rem != 0) & (half == 0), 1, wf0)
    wf1 = jnp.where((rem != 0) & (half == 1), 1, wf1)

    @pl.when(wf0 == 1)
    def _():
        _wait_half(0, sc_sem0)

    @pl.when(wf1 == 1)
    def _():
        _wait_half(1, sc_sem1)


def kernel(x, weight):
    wt3 = weight.T.reshape(8, 8, VOCAB)
    wtail = jnp.pad(weight[MAIN_V:].T, ((0, 0), (0, 2 * CH - TAIL)))
    wtl3 = wtail.reshape(8, 8, 2 * CH)
    out128 = _embed(x.astype(jnp.int32), wt3, wtl3)
    return out128[:, :DIM]
